# baseline (speedup 1.0000x reference)
import jax
import jax.numpy as jnp
import numpy as np
from jax import lax
from jax.experimental import pallas as pl
from jax.experimental.pallas import tpu as pltpu
from jax.experimental.pallas import tpu_sc as plsc

N = 10000
NP = 10240
E = 160000
LANES = 16
NSUB = 16
NCORE = 2
EPT = E // NSUB
EB = 80
NBATCH = EPT // EB
RPT = NP // NSUB
CW = 128
NRING = 4


_GATHER_DNUMS = lax.GatherDimensionNumbers(
    offset_dims=(), collapsed_slice_dims=(0,), start_index_map=(0,))


def _splat(vec, j):
    idx = jnp.full((LANES, 1), j, jnp.int32)
    return lax.gather(vec, idx, _GATHER_DNUMS, (1,),
                      mode=lax.GatherScatterMode.PROMISE_IN_BOUNDS)


def _scale_rows(rbf, rout, ewv):
    def group(g, carry):
        ewreg = ewv[pl.ds(g * LANES, LANES)]
        for j in range(LANES):
            sp = _splat(ewreg, j)
            i = g * LANES + j
            for k in range(CW // 32):
                w = rbf[i, pl.ds(k * LANES, LANES)]
                a = lax.bitcast_convert_type(w << 16, jnp.float32)
                b = lax.bitcast_convert_type(w & jnp.int32(-65536),
                                             jnp.float32)
                rout[i, pl.ds(k * 32, LANES)] = a * sp
                rout[i, pl.ds(k * 32 + LANES, LANES)] = b * sp
        return carry
    lax.fori_loop(0, EB // LANES, group, 0)


def _one_pass(xc, outc, src_r, dst_r, ew_r, acc, rows, srow, srcv, dstv,
              ewv, gs, isem, ss, sid):
    row0 = sid * RPT
    ebase = sid * EPT

    def zfill(i, carry):
        for k in range(CW // LANES):
            srow[0][i, pl.ds(k * LANES, LANES)] = jnp.zeros(
                (LANES,), jnp.float32)
        return carry
    lax.fori_loop(0, EB, zfill, 0)
    for z in range(RPT // EB):
        pltpu.async_copy(srow[0], acc.at[pl.ds(row0 + z * EB, EB)],
                         gs[z % 2])
    for z in range(RPT // EB):
        pltpu.make_async_copy(
            srow[0], acc.at[pl.ds(row0 + (z % 2) * EB, EB)],
            gs[z % 2]).wait()
    plsc.subcore_barrier()

    def idx_descs(b, p):
        sl = pl.ds(ebase + b * EB, EB)
        return (pltpu.make_async_copy(src_r.at[sl], srcv[p], isem[p]),
                pltpu.make_async_copy(dst_r.at[sl], dstv[p], isem[p]),
                pltpu.make_async_copy(ew_r.at[sl], ewv[p], isem[p]))

    def idx_start(b, p):
        for d in idx_descs(b, p):
            d.start()

    def idx_wait(b, p):
        for d in idx_descs(b, p):
            d.wait()

    def gather_desc(p):
        return pltpu.make_async_copy(xc.at[srcv[p]], rows[p], gs[p])

    def scat_start(p, q):
        pltpu.async_copy(srow[q], acc.at[dstv[p]], ss[q], add=True)

    def scat_wait(p, q):
        pltpu.make_async_copy(srow[q], acc.at[dstv[p]], ss[q]).wait()

    def substep(k, p, w_scat, g_next, i_next2):
        q = p % 2
        if w_scat:
            scat_wait((p + 2) % NRING, q)
        if g_next:
            idx_wait(k + 1, (p + 1) % NRING)
            gather_desc((p + 1) % NRING).start()
        if i_next2:
            idx_start(k + 2, (p + 2) % NRING)
        gather_desc(p).wait()
        _scale_rows(rows[p], srow[q], ewv[p])
        scat_start(p, q)

    idx_start(0, 0)
    idx_start(1, 1)
    idx_wait(0, 0)
    gather_desc(0).start()
    substep(0, 0, False, True, True)
    substep(1, 1, False, True, True)
    substep(2, 2, True, True, True)
    substep(3, 3, True, True, True)

    def step(t, carry):
        a = NRING * t
        for j in range(NRING):
            substep(a + j, j, True, True, True)
        return carry
    lax.fori_loop(1, (NBATCH - 5) // NRING, step, 0)

    substep(120, 0, True, True, True)
    substep(121, 1, True, True, True)
    substep(122, 2, True, True, True)
    substep(123, 3, True, True, False)
    substep(124, 0, True, False, False)
    scat_wait(3, 1)
    scat_wait(0, 0)

    plsc.subcore_barrier()
    pltpu.sync_copy(acc.at[pl.ds(row0, RPT)], outc.at[pl.ds(row0, RPT)])
    plsc.subcore_barrier()


def _segment_sum_sc(xstack, src, dst, ew):
    nc = xstack.shape[0]
    npc = nc // NCORE
    mesh = plsc.VectorSubcoreMesh(core_axis_name="c", subcore_axis_name="s")
    out_type = jax.ShapeDtypeStruct((nc, NP, CW), jnp.float32)
    scratch = [
        pltpu.VMEM_SHARED((NP, CW), jnp.float32),
    ]
    scratch += [pltpu.VMEM((EB, CW // 2), jnp.int32) for _ in range(NRING)]
    scratch += [pltpu.VMEM((EB, CW), jnp.float32) for _ in range(2)]
    scratch += [pltpu.VMEM((EB,), jnp.int32) for _ in range(NRING)]
    scratch += [pltpu.VMEM((EB,), jnp.int32) for _ in range(NRING)]
    scratch += [pltpu.VMEM((EB,), jnp.float32) for _ in range(NRING)]
    scratch += [pltpu.SemaphoreType.DMA] * (2 * NRING + 2 + 2)

    def body(xs_r, src_r, dst_r, ew_r, out_r, *sc):
        acc = sc[0]
        rows = sc[1:1 + NRING]
        srow = sc[1 + NRING:3 + NRING]
        srcv = sc[3 + NRING:3 + 2 * NRING]
        dstv = sc[3 + 2 * NRING:3 + 3 * NRING]
        ewv = sc[3 + 3 * NRING:3 + 4 * NRING]
        sems = sc[3 + 4 * NRING:]
        gs = sems[:NRING]
        isem = sems[NRING:2 * NRING]
        ss = sems[2 * NRING:2 * NRING + 2]
        cid = lax.axis_index("c")
        sid = lax.axis_index("s")

        def chunk_pass(j, carry):
            chunk = cid * npc + j
            _one_pass(xs_r.at[chunk], out_r.at[chunk], src_r, dst_r,
                      ew_r, acc, rows, srow, srcv, dstv, ewv, gs, isem,
                      ss, sid)
            return carry
        lax.fori_loop(0, npc, chunk_pass, 0)

    f = pl.kernel(body, out_type=out_type, mesh=mesh, scratch_types=scratch,
                  compiler_params=pltpu.CompilerParams(
                      use_tc_tiling_on_sc=False))
    return f(xstack, src, dst, ew)


BN = 2048


def _layer_mm(aggr, xstack, Wrel, Wroot, brel):
    nci = aggr.shape[0]
    nco = 4
    din = nci * CW

    def body(aggr_ref, x_ref, wrel, wroot, b, out_ref, obf_ref):
        acc = jnp.broadcast_to(b[...], (BN, 512)).astype(jnp.float32)
        for c in range(nci):
            acc = acc + jnp.dot(aggr_ref[c], wrel[pl.ds(c * CW, CW), :],
                                preferred_element_type=jnp.float32)
            acc = acc + jnp.dot(x_ref[c], wroot[pl.ds(c * CW, CW), :],
                                preferred_element_type=jnp.float32)
        acc = jnp.where(acc >= 0, acc, acc * jnp.float32(0.01))
        for c in range(nco):
            out_ref[c] = acc[:, c * CW:(c + 1) * CW]
            obf_ref[c] = acc[:, c * CW:(c + 1) * CW].astype(jnp.bfloat16)

    grid = (NP // BN,)
    in_specs = [
        pl.BlockSpec((nci, BN, CW), lambda i: (0, i, 0)),
        pl.BlockSpec((nci, BN, CW), lambda i: (0, i, 0)),
        pl.BlockSpec((din, 512), lambda i: (0, 0)),
        pl.BlockSpec((din, 512), lambda i: (0, 0)),
        pl.BlockSpec((1, 512), lambda i: (0, 0)),
    ]
    out_specs = (pl.BlockSpec((nco, BN, CW), lambda i: (0, i, 0)),
                 pl.BlockSpec((nco, BN, CW), lambda i: (0, i, 0)))
    out_shape = (jax.ShapeDtypeStruct((nco, NP, CW), jnp.float32),
                 jax.ShapeDtypeStruct((nco, NP, CW), jnp.bfloat16))
    f = pl.pallas_call(body, grid=grid, in_specs=in_specs,
                       out_specs=out_specs, out_shape=out_shape)
    return f(aggr, xstack, Wrel, Wroot, brel.reshape(1, 512))


def _head(xstack, W1, b1, W2, b2):
    def body(x_ref, w1, b1r, w2, b2r, out):
        acc = jnp.broadcast_to(b1r[...], (BN, 512)).astype(jnp.float32)
        for c in range(4):
            acc = acc + jnp.dot(x_ref[c], w1[pl.ds(c * CW, CW), :],
                                preferred_element_type=jnp.float32)
        h = jnp.maximum(acc, 0.0)
        out[...] = (jnp.dot(h, w2[...], preferred_element_type=jnp.float32)
                    + b2r[...])

    grid = (NP // BN,)
    in_specs = [
        pl.BlockSpec((4, BN, CW), lambda i: (0, i, 0)),
        pl.BlockSpec((512, 512), lambda i: (0, 0)),
        pl.BlockSpec((1, 512), lambda i: (0, 0)),
        pl.BlockSpec((512, 128), lambda i: (0, 0)),
        pl.BlockSpec((1, 128), lambda i: (0, 0)),
    ]
    out_specs = pl.BlockSpec((BN, 128), lambda i: (i, 0))
    out_shape = jax.ShapeDtypeStruct((NP, 128), jnp.float32)
    f = pl.pallas_call(body, grid=grid, in_specs=in_specs,
                       out_specs=out_specs, out_shape=out_shape)
    return f(xstack, W1, b1.reshape(1, 512), W2, b2.reshape(1, 128))


def kernel(x_Entity, x_Transaction, edge_index_e2t, edge_index_t2e,
           edge_weight_e2t, edge_weight_t2e,
           l1_e2t_Wrel, l1_e2t_brel, l1_e2t_Wroot,
           l1_t2e_Wrel, l1_t2e_brel, l1_t2e_Wroot,
           l2_e2t_Wrel, l2_e2t_brel, l2_e2t_Wroot,
           l2_t2e_Wrel, l2_t2e_brel, l2_t2e_Wroot,
           l3_e2t_Wrel, l3_e2t_brel, l3_e2t_Wroot,
           l3_t2e_Wrel, l3_t2e_brel, l3_t2e_Wroot,
           lin1_W, lin1_b, lin2_W, lin2_b):
    src_e2t = edge_index_e2t[0].astype(jnp.int32)
    dst_e2t = edge_index_e2t[1].astype(jnp.int32)
    src_t2e = edge_index_t2e[0].astype(jnp.int32)
    dst_t2e = edge_index_t2e[1].astype(jnp.int32)
    ew_e2t = edge_weight_e2t.astype(jnp.float32)
    ew_t2e = edge_weight_t2e.astype(jnp.float32)

    def stack2(x):
        xp = jnp.pad(x, ((0, NP - N), (0, 0)))
        return jnp.stack([xp[:, :CW], xp[:, CW:]])

    xe = stack2(x_Entity)
    xt = stack2(x_Transaction)

    def pack_i32(xb):
        nc = xb.shape[0]
        return lax.bitcast_convert_type(
            xb.reshape(nc, NP, CW // 2, 2), jnp.int32)

    xe_b = pack_i32(xe.astype(jnp.bfloat16))
    xt_b = pack_i32(xt.astype(jnp.bfloat16))

    p32 = np.concatenate([np.arange(0, 32, 2), np.arange(1, 32, 2)])

    def perm(w):
        j = np.arange(w.shape[0])
        return w[(j // 32) * 32 + p32[j % 32]]

    wts = {
        1: (l1_e2t_Wrel, l1_e2t_brel, l1_e2t_Wroot,
            l1_t2e_Wrel, l1_t2e_brel, l1_t2e_Wroot),
        2: (l2_e2t_Wrel, l2_e2t_brel, l2_e2t_Wroot,
            l2_t2e_Wrel, l2_t2e_brel, l2_t2e_Wroot),
        3: (l3_e2t_Wrel, l3_e2t_brel, l3_e2t_Wroot,
            l3_t2e_Wrel, l3_t2e_brel, l3_t2e_Wroot),
    }

    for l in (1, 2, 3):
        (wrel_et, brel_et, wroot_et,
         wrel_te, brel_te, wroot_te) = wts[l]
        if l < 3:
            aggr_t = _segment_sum_sc(xe_b, src_e2t, dst_e2t, ew_e2t)
        aggr_e = _segment_sum_sc(xt_b, src_t2e, dst_t2e, ew_t2e)
        new_xe, new_xe_b = _layer_mm(aggr_e, xe, perm(wrel_te), wroot_te,
                                     brel_te)
        if l < 3:
            xt, xt_bf = _layer_mm(aggr_t, xt, perm(wrel_et), wroot_et,
                                  brel_et)
            xt_b = pack_i32(xt_bf)
        xe, xe_b = new_xe, pack_i32(new_xe_b)

    return _head(xe, lin1_W, lin1_b, lin2_W, lin2_b)[:N]


# speedup vs baseline: 2.2371x; 2.2371x over previous
import jax
import jax.numpy as jnp
from jax import lax
from jax.experimental import pallas as pl
from jax.experimental.pallas import tpu as pltpu
from jax.experimental.pallas import tpu_sc as plsc

N = 10000
NP = 10240
E = 160000
LANES = 16
NSUB = 16
NCORE = 2
EPT = E // NSUB
EB = 80
NBATCH = EPT // EB
RPT = NP // NSUB
CW = 128
NRING = 4


_GATHER_DNUMS = lax.GatherDimensionNumbers(
    offset_dims=(), collapsed_slice_dims=(0,), start_index_map=(0,))


def _splat(vec, j):
    idx = jnp.full((LANES, 1), j, jnp.int32)
    return lax.gather(vec, idx, _GATHER_DNUMS, (1,),
                      mode=lax.GatherScatterMode.PROMISE_IN_BOUNDS)


def _scale_rows(rows, ewv):
    def group(g, carry):
        ewreg = ewv[pl.ds(g * LANES, LANES)]
        for j in range(LANES):
            sp = _splat(ewreg, j)
            i = g * LANES + j
            for k in range(CW // LANES):
                sl = pl.ds(k * LANES, LANES)
                rows[i, sl] = rows[i, sl] * sp
        return carry
    lax.fori_loop(0, EB // LANES, group, 0)


def _one_pass(xc, outc, idx_r, ew_r, acc, rows, ibuf, ewv,
              gs, isem, ss, sid):
    row0 = sid * RPT
    ebase = sid * EPT

    def zfill(i, carry):
        for k in range(CW // LANES):
            rows[0][i, pl.ds(k * LANES, LANES)] = jnp.zeros(
                (LANES,), jnp.float32)
        return carry
    lax.fori_loop(0, EB, zfill, 0)
    for z in range(RPT // EB):
        pltpu.async_copy(rows[0], acc.at[pl.ds(row0 + z * EB, EB)],
                         gs[z % 2])
    for z in range(RPT // EB):
        pltpu.make_async_copy(
            rows[0], acc.at[pl.ds(row0 + (z % 2) * EB, EB)],
            gs[z % 2]).wait()
    plsc.subcore_barrier()

    def idx_descs(b, p):
        slab = sid * NBATCH + b
        return (pltpu.make_async_copy(idx_r.at[slab], ibuf[p], isem[p]),
                pltpu.make_async_copy(
                    ew_r.at[pl.ds(ebase + b * EB, EB)], ewv[p], isem[p]))

    def idx_start(b, p):
        for d in idx_descs(b, p):
            d.start()

    def idx_wait(b, p):
        for d in idx_descs(b, p):
            d.wait()

    def gather_desc(p):
        return pltpu.make_async_copy(xc.at[ibuf[p].at[0]], rows[p], gs[p])

    def scat_start(p):
        pltpu.async_copy(rows[p], acc.at[ibuf[p].at[1]], ss[p], add=True)

    def scat_wait(p):
        pltpu.make_async_copy(rows[p], acc.at[ibuf[p].at[1]], ss[p]).wait()

    def substep(k, p, w_scat, g_next, i_next2):
        if w_scat:
            scat_wait((p + 2) % NRING)
        if g_next:
            idx_wait(k + 1, (p + 1) % NRING)
            gather_desc((p + 1) % NRING).start()
        if i_next2:
            idx_start(k + 2, (p + 2) % NRING)
        gather_desc(p).wait()
        _scale_rows(rows[p], ewv[p])
        scat_start(p)

    idx_start(0, 0)
    idx_start(1, 1)
    idx_wait(0, 0)
    gather_desc(0).start()
    substep(0, 0, False, True, True)
    substep(1, 1, False, True, True)
    substep(2, 2, True, True, True)
    substep(3, 3, True, True, True)

    def step(t, carry):
        a = NRING * t
        for j in range(NRING):
            substep(a + j, j, True, True, True)
        return carry
    lax.fori_loop(1, (NBATCH - 5) // NRING, step, 0)

    substep(120, 0, True, True, True)
    substep(121, 1, True, True, True)
    substep(122, 2, True, True, True)
    substep(123, 3, True, True, False)
    substep(124, 0, True, False, False)
    scat_wait(3)
    scat_wait(0)

    plsc.subcore_barrier()
    pltpu.sync_copy(acc.at[pl.ds(row0, RPT)], outc.at[pl.ds(row0, RPT)])
    plsc.subcore_barrier()


def _segment_sum_sc(xstack, idx2, ew):
    nc = xstack.shape[0]
    npc = nc // NCORE
    mesh = plsc.VectorSubcoreMesh(core_axis_name="c", subcore_axis_name="s")
    out_type = jax.ShapeDtypeStruct((nc, NP, CW), jnp.float32)
    scratch = [
        pltpu.VMEM_SHARED((NP, CW), jnp.float32),
    ]
    scratch += [pltpu.VMEM((EB, CW), jnp.float32) for _ in range(NRING)]
    scratch += [pltpu.VMEM((2, EB), jnp.int32) for _ in range(NRING)]
    scratch += [pltpu.VMEM((EB,), jnp.float32) for _ in range(NRING)]
    scratch += [pltpu.SemaphoreType.DMA] * (3 * NRING)

    def body(xs_r, idx_r, ew_r, out_r, *sc):
        acc = sc[0]
        rows = sc[1:1 + NRING]
        ibuf = sc[1 + NRING:1 + 2 * NRING]
        ewv = sc[1 + 2 * NRING:1 + 3 * NRING]
        sems = sc[1 + 3 * NRING:]
        gs = sems[:NRING]
        isem = sems[NRING:2 * NRING]
        ss = sems[2 * NRING:3 * NRING]
        cid = lax.axis_index("c")
        sid = lax.axis_index("s")

        def chunk_pass(j, carry):
            chunk = cid * npc + j
            _one_pass(xs_r.at[chunk], out_r.at[chunk], idx_r,
                      ew_r, acc, rows, ibuf, ewv, gs, isem, ss,
                      sid)
            return carry
        lax.fori_loop(0, npc, chunk_pass, 0)

    f = pl.kernel(body, out_type=out_type, mesh=mesh, scratch_types=scratch)
    return f(xstack, idx2, ew)


BN = 2048


def _layer_mm(aggr, xstack, Wrel, Wroot, brel):
    nci = aggr.shape[0]
    nco = 4
    din = nci * CW

    def body(aggr_ref, x_ref, wrel, wroot, b, out_ref):
        acc = jnp.broadcast_to(b[...], (BN, 512)).astype(jnp.float32)
        for c in range(nci):
            acc = acc + jnp.dot(aggr_ref[c], wrel[pl.ds(c * CW, CW), :],
                                preferred_element_type=jnp.float32)
            acc = acc + jnp.dot(x_ref[c], wroot[pl.ds(c * CW, CW), :],
                                preferred_element_type=jnp.float32)
        acc = jnp.where(acc >= 0, acc, acc * jnp.float32(0.01))
        for c in range(nco):
            out_ref[c] = acc[:, c * CW:(c + 1) * CW]

    grid = (NP // BN,)
    in_specs = [
        pl.BlockSpec((nci, BN, CW), lambda i: (0, i, 0)),
        pl.BlockSpec((nci, BN, CW), lambda i: (0, i, 0)),
        pl.BlockSpec((din, 512), lambda i: (0, 0)),
        pl.BlockSpec((din, 512), lambda i: (0, 0)),
        pl.BlockSpec((1, 512), lambda i: (0, 0)),
    ]
    out_specs = pl.BlockSpec((nco, BN, CW), lambda i: (0, i, 0))
    out_shape = jax.ShapeDtypeStruct((nco, NP, CW), jnp.float32)
    f = pl.pallas_call(body, grid=grid, in_specs=in_specs,
                       out_specs=out_specs, out_shape=out_shape)
    return f(aggr, xstack, Wrel, Wroot, brel.reshape(1, 512))


def _head(xstack, W1, b1, W2, b2):
    def body(x_ref, w1, b1r, w2, b2r, out):
        acc = jnp.broadcast_to(b1r[...], (BN, 512)).astype(jnp.float32)
        for c in range(4):
            acc = acc + jnp.dot(x_ref[c], w1[pl.ds(c * CW, CW), :],
                                preferred_element_type=jnp.float32)
        h = jnp.maximum(acc, 0.0)
        out[...] = (jnp.dot(h, w2[...], preferred_element_type=jnp.float32)
                    + b2r[...])

    grid = (NP // BN,)
    in_specs = [
        pl.BlockSpec((4, BN, CW), lambda i: (0, i, 0)),
        pl.BlockSpec((512, 512), lambda i: (0, 0)),
        pl.BlockSpec((1, 512), lambda i: (0, 0)),
        pl.BlockSpec((512, 128), lambda i: (0, 0)),
        pl.BlockSpec((1, 128), lambda i: (0, 0)),
    ]
    out_specs = pl.BlockSpec((BN, 128), lambda i: (i, 0))
    out_shape = jax.ShapeDtypeStruct((NP, 128), jnp.float32)
    f = pl.pallas_call(body, grid=grid, in_specs=in_specs,
                       out_specs=out_specs, out_shape=out_shape)
    return f(xstack, W1, b1.reshape(1, 512), W2, b2.reshape(1, 128))


def kernel(x_Entity, x_Transaction, edge_index_e2t, edge_index_t2e,
           edge_weight_e2t, edge_weight_t2e,
           l1_e2t_Wrel, l1_e2t_brel, l1_e2t_Wroot,
           l1_t2e_Wrel, l1_t2e_brel, l1_t2e_Wroot,
           l2_e2t_Wrel, l2_e2t_brel, l2_e2t_Wroot,
           l2_t2e_Wrel, l2_t2e_brel, l2_t2e_Wroot,
           l3_e2t_Wrel, l3_e2t_brel, l3_e2t_Wroot,
           l3_t2e_Wrel, l3_t2e_brel, l3_t2e_Wroot,
           lin1_W, lin1_b, lin2_W, lin2_b):
    def slabs(ei):
        return jnp.stack([ei[0].astype(jnp.int32).reshape(-1, EB),
                          ei[1].astype(jnp.int32).reshape(-1, EB)], axis=1)

    idx_e2t = slabs(edge_index_e2t)
    idx_t2e = slabs(edge_index_t2e)
    ew_e2t = edge_weight_e2t.astype(jnp.float32)
    ew_t2e = edge_weight_t2e.astype(jnp.float32)

    def stack2(x):
        xp = jnp.pad(x, ((0, NP - N), (0, 0)))
        return jnp.stack([xp[:, :CW], xp[:, CW:]])

    xe = stack2(x_Entity)
    xt = stack2(x_Transaction)

    wts = {
        1: (l1_e2t_Wrel, l1_e2t_brel, l1_e2t_Wroot,
            l1_t2e_Wrel, l1_t2e_brel, l1_t2e_Wroot),
        2: (l2_e2t_Wrel, l2_e2t_brel, l2_e2t_Wroot,
            l2_t2e_Wrel, l2_t2e_brel, l2_t2e_Wroot),
        3: (l3_e2t_Wrel, l3_e2t_brel, l3_e2t_Wroot,
            l3_t2e_Wrel, l3_t2e_brel, l3_t2e_Wroot),
    }

    for l in (1, 2, 3):
        (wrel_et, brel_et, wroot_et,
         wrel_te, brel_te, wroot_te) = wts[l]
        if l < 3:
            aggr_t = _segment_sum_sc(xe, idx_e2t, ew_e2t)
        aggr_e = _segment_sum_sc(xt, idx_t2e, ew_t2e)
        new_xe = _layer_mm(aggr_e, xe, wrel_te, wroot_te, brel_te)
        if l < 3:
            xt = _layer_mm(aggr_t, xt, wrel_et, wroot_et, brel_et)
        xe = new_xe

    return _head(xe, lin1_W, lin1_b, lin2_W, lin2_b)[:N]


# speedup vs baseline: 2.2628x; 1.0115x over previous
import jax
import jax.numpy as jnp
from jax import lax
from jax.experimental import pallas as pl
from jax.experimental.pallas import tpu as pltpu
from jax.experimental.pallas import tpu_sc as plsc

N = 10000
NP = 10240
E = 160000
LANES = 16
NSUB = 16
NCORE = 2
EPT = E // NSUB
EB = 80
NBATCH = EPT // EB
RPT = NP // NSUB
CW = 128
NRING = 4


_GATHER_DNUMS = lax.GatherDimensionNumbers(
    offset_dims=(), collapsed_slice_dims=(0,), start_index_map=(0,))


def _splat(vec, j):
    idx = jnp.full((LANES, 1), j, jnp.int32)
    return lax.gather(vec, idx, _GATHER_DNUMS, (1,),
                      mode=lax.GatherScatterMode.PROMISE_IN_BOUNDS)


def _scale_rows(rows, ewv):
    def group(g, carry):
        ewreg = ewv[pl.ds(g * LANES, LANES)]
        for j in range(LANES):
            sp = _splat(ewreg, j)
            i = g * LANES + j
            for k in range(CW // LANES):
                sl = pl.ds(k * LANES, LANES)
                rows[i, sl] = rows[i, sl] * sp
        return carry
    lax.fori_loop(0, EB // LANES, group, 0)


def _one_pass(xc, outc, src_r, dst_r, ew_r, acc, rows, srcv, dstv, ewv,
              gs, isem, ss, sid):
    row0 = sid * RPT
    ebase = sid * EPT

    def zfill(i, carry):
        for k in range(CW // LANES):
            rows[0][i, pl.ds(k * LANES, LANES)] = jnp.zeros(
                (LANES,), jnp.float32)
        return carry
    lax.fori_loop(0, EB, zfill, 0)
    for z in range(RPT // EB):
        pltpu.async_copy(rows[0], acc.at[pl.ds(row0 + z * EB, EB)],
                         gs[z % 2])
    for z in range(RPT // EB):
        pltpu.make_async_copy(
            rows[0], acc.at[pl.ds(row0 + (z % 2) * EB, EB)],
            gs[z % 2]).wait()
    plsc.subcore_barrier()

    def idx_descs(b, p):
        sl = pl.ds(ebase + b * EB, EB)
        return (pltpu.make_async_copy(src_r.at[sl], srcv[p], isem[p]),
                pltpu.make_async_copy(dst_r.at[sl], dstv[p], isem[p]),
                pltpu.make_async_copy(ew_r.at[sl], ewv[p], isem[p]))

    def idx_start(b, p):
        for d in idx_descs(b, p):
            d.start()

    def idx_wait(b, p):
        for d in idx_descs(b, p):
            d.wait()

    def gather_desc(p):
        return pltpu.make_async_copy(xc.at[srcv[p]], rows[p], gs[p])

    def scat_start(p):
        pltpu.async_copy(rows[p], acc.at[dstv[p]], ss[p], add=True)

    def scat_wait(p):
        pltpu.make_async_copy(rows[p], acc.at[dstv[p]], ss[p]).wait()

    def substep(k, p, w_scat, g_next, i_next2):
        if w_scat:
            scat_wait((p + 2) % NRING)
        if g_next:
            idx_wait(k + 1, (p + 1) % NRING)
            gather_desc((p + 1) % NRING).start()
        if i_next2:
            idx_start(k + 2, (p + 2) % NRING)
        gather_desc(p).wait()
        _scale_rows(rows[p], ewv[p])
        scat_start(p)

    idx_start(0, 0)
    idx_start(1, 1)
    idx_wait(0, 0)
    gather_desc(0).start()
    substep(0, 0, False, True, True)
    substep(1, 1, False, True, True)
    substep(2, 2, True, True, True)
    substep(3, 3, True, True, True)

    def step(t, carry):
        a = NRING * t
        for j in range(NRING):
            substep(a + j, j, True, True, True)
        return carry
    lax.fori_loop(1, (NBATCH - 5) // NRING, step, 0)

    substep(120, 0, True, True, True)
    substep(121, 1, True, True, True)
    substep(122, 2, True, True, True)
    substep(123, 3, True, True, False)
    substep(124, 0, True, False, False)
    scat_wait(3)
    scat_wait(0)

    plsc.subcore_barrier()
    pltpu.sync_copy(acc.at[pl.ds(row0, RPT)], outc.at[pl.ds(row0, RPT)])
    plsc.subcore_barrier()


def _segment_sum_sc(xstack, src, dst, ew):
    nc = xstack.shape[0]
    npc = nc // NCORE
    mesh = plsc.VectorSubcoreMesh(core_axis_name="c", subcore_axis_name="s")
    out_type = jax.ShapeDtypeStruct((nc, NP, CW), jnp.float32)
    scratch = [
        pltpu.VMEM_SHARED((NP, CW), jnp.float32),
    ]
    scratch += [pltpu.VMEM((EB, CW), jnp.float32) for _ in range(NRING)]
    scratch += [pltpu.VMEM((EB,), jnp.int32) for _ in range(NRING)]
    scratch += [pltpu.VMEM((EB,), jnp.int32) for _ in range(NRING)]
    scratch += [pltpu.VMEM((EB,), jnp.float32) for _ in range(NRING)]
    scratch += [pltpu.SemaphoreType.DMA] * (3 * NRING)

    def body(xs_r, src_r, dst_r, ew_r, out_r, *sc):
        acc = sc[0]
        rows = sc[1:1 + NRING]
        srcv = sc[1 + NRING:1 + 2 * NRING]
        dstv = sc[1 + 2 * NRING:1 + 3 * NRING]
        ewv = sc[1 + 3 * NRING:1 + 4 * NRING]
        sems = sc[1 + 4 * NRING:]
        gs = sems[:NRING]
        isem = sems[NRING:2 * NRING]
        ss = sems[2 * NRING:3 * NRING]
        cid = lax.axis_index("c")
        sid = lax.axis_index("s")

        def chunk_pass(j, carry):
            chunk = cid * npc + j
            _one_pass(xs_r.at[chunk], out_r.at[chunk], src_r, dst_r,
                      ew_r, acc, rows, srcv, dstv, ewv, gs, isem, ss,
                      sid)
            return carry
        lax.fori_loop(0, npc, chunk_pass, 0)

    f = pl.kernel(body, out_type=out_type, mesh=mesh, scratch_types=scratch)
    return f(xstack, src, dst, ew)


BN = 2048


def _layer_mm(aggr, xstack, Wrel, Wroot, brel):
    nci = aggr.shape[0]
    nco = 4
    din = nci * CW

    def body(aggr_ref, x_ref, wrel, wroot, b, out_ref):
        acc = jnp.broadcast_to(b[...], (BN, 512)).astype(jnp.float32)
        for c in range(nci):
            acc = acc + jnp.dot(aggr_ref[c], wrel[pl.ds(c * CW, CW), :],
                                preferred_element_type=jnp.float32)
            acc = acc + jnp.dot(x_ref[c], wroot[pl.ds(c * CW, CW), :],
                                preferred_element_type=jnp.float32)
        acc = jnp.where(acc >= 0, acc, acc * jnp.float32(0.01))
        for c in range(nco):
            out_ref[c] = acc[:, c * CW:(c + 1) * CW]

    grid = (NP // BN,)
    in_specs = [
        pl.BlockSpec((nci, BN, CW), lambda i: (0, i, 0)),
        pl.BlockSpec((nci, BN, CW), lambda i: (0, i, 0)),
        pl.BlockSpec((din, 512), lambda i: (0, 0)),
        pl.BlockSpec((din, 512), lambda i: (0, 0)),
        pl.BlockSpec((1, 512), lambda i: (0, 0)),
    ]
    out_specs = pl.BlockSpec((nco, BN, CW), lambda i: (0, i, 0))
    out_shape = jax.ShapeDtypeStruct((nco, NP, CW), jnp.float32)
    f = pl.pallas_call(body, grid=grid, in_specs=in_specs,
                       out_specs=out_specs, out_shape=out_shape)
    return f(aggr, xstack, Wrel, Wroot, brel.reshape(1, 512))


def _head(xstack, W1, b1, W2, b2):
    def body(x_ref, w1, b1r, w2, b2r, out):
        acc = jnp.broadcast_to(b1r[...], (BN, 512)).astype(jnp.float32)
        for c in range(4):
            acc = acc + jnp.dot(x_ref[c], w1[pl.ds(c * CW, CW), :],
                                preferred_element_type=jnp.float32)
        h = jnp.maximum(acc, 0.0)
        out[...] = (jnp.dot(h, w2[...], preferred_element_type=jnp.float32)
                    + b2r[...])

    grid = (NP // BN,)
    in_specs = [
        pl.BlockSpec((4, BN, CW), lambda i: (0, i, 0)),
        pl.BlockSpec((512, 512), lambda i: (0, 0)),
        pl.BlockSpec((1, 512), lambda i: (0, 0)),
        pl.BlockSpec((512, 128), lambda i: (0, 0)),
        pl.BlockSpec((1, 128), lambda i: (0, 0)),
    ]
    out_specs = pl.BlockSpec((BN, 128), lambda i: (i, 0))
    out_shape = jax.ShapeDtypeStruct((NP, 128), jnp.float32)
    f = pl.pallas_call(body, grid=grid, in_specs=in_specs,
                       out_specs=out_specs, out_shape=out_shape)
    return f(xstack, W1, b1.reshape(1, 512), W2, b2.reshape(1, 128))


def kernel(x_Entity, x_Transaction, edge_index_e2t, edge_index_t2e,
           edge_weight_e2t, edge_weight_t2e,
           l1_e2t_Wrel, l1_e2t_brel, l1_e2t_Wroot,
           l1_t2e_Wrel, l1_t2e_brel, l1_t2e_Wroot,
           l2_e2t_Wrel, l2_e2t_brel, l2_e2t_Wroot,
           l2_t2e_Wrel, l2_t2e_brel, l2_t2e_Wroot,
           l3_e2t_Wrel, l3_e2t_brel, l3_e2t_Wroot,
           l3_t2e_Wrel, l3_t2e_brel, l3_t2e_Wroot,
           lin1_W, lin1_b, lin2_W, lin2_b):
    src_e2t = edge_index_e2t[0].astype(jnp.int32)
    dst_e2t = edge_index_e2t[1].astype(jnp.int32)
    src_t2e = edge_index_t2e[0].astype(jnp.int32)
    dst_t2e = edge_index_t2e[1].astype(jnp.int32)
    ew_e2t = edge_weight_e2t.astype(jnp.float32)
    ew_t2e = edge_weight_t2e.astype(jnp.float32)

    def stack2(x):
        xp = jnp.pad(x, ((0, NP - N), (0, 0)))
        return jnp.stack([xp[:, :CW], xp[:, CW:]])

    xe = stack2(x_Entity)
    xt = stack2(x_Transaction)

    wts = {
        1: (l1_e2t_Wrel, l1_e2t_brel, l1_e2t_Wroot,
            l1_t2e_Wrel, l1_t2e_brel, l1_t2e_Wroot),
        2: (l2_e2t_Wrel, l2_e2t_brel, l2_e2t_Wroot,
            l2_t2e_Wrel, l2_t2e_brel, l2_t2e_Wroot),
        3: (l3_e2t_Wrel, l3_e2t_brel, l3_e2t_Wroot,
            l3_t2e_Wrel, l3_t2e_brel, l3_t2e_Wroot),
    }

    for l in (1, 2, 3):
        (wrel_et, brel_et, wroot_et,
         wrel_te, brel_te, wroot_te) = wts[l]
        if l < 3:
            aggr_t = _segment_sum_sc(xe, src_e2t, dst_e2t, ew_e2t)
        aggr_e = _segment_sum_sc(xt, src_t2e, dst_t2e, ew_t2e)
        new_xe = _layer_mm(aggr_e, xe, wrel_te, wroot_te, brel_te)
        if l < 3:
            xt = _layer_mm(aggr_t, xt, wrel_et, wroot_et, brel_et)
        xe = new_xe

    return _head(xe, lin1_W, lin1_b, lin2_W, lin2_b)[:N]
